# baseline (speedup 1.0000x reference)
import functools

import jax
import jax.numpy as jnp
from jax import lax
from jax.experimental import pallas as pl
from jax.experimental.pallas import tpu as pltpu
from jax.experimental.pallas import tpu_sc as plsc

_NC = 2
_NS = 16
_NW = _NC * _NS
_LANES = 16


def _gather_body(rows_per_w, embed_dim, table_hbm, idx_hbm, out_hbm,
                 idx_v, out_v, sem):
    wid = lax.axis_index("s") * _NC + lax.axis_index("c")
    base = wid * rows_per_w

    pltpu.sync_copy(idx_hbm.at[pl.ds(base, rows_per_w)], idx_v)

    width = idx_v.shape[1]
    chunks = width // _LANES
    lane_iota = lax.iota(jnp.int32, _LANES)

    def row_step(r, carry):
        for c in range(chunks):
            col0 = (c * _LANES) % embed_dim
            v = idx_v[r, pl.ds(c * _LANES, _LANES)]
            idx_v[r, pl.ds(c * _LANES, _LANES)] = (
                v * embed_dim + (lane_iota + col0))
        pltpu.async_copy(table_hbm.at[idx_v.at[r]], out_v.at[r], sem)
        return carry

    lax.fori_loop(0, rows_per_w, row_step, 0, unroll=False)

    pltpu.make_async_copy(out_hbm.at[pl.ds(base, rows_per_w)], out_v, sem).wait()

    pltpu.sync_copy(out_v, out_hbm.at[pl.ds(base, rows_per_w)])


def kernel(input, index):
    vocab, embed_dim = input.shape
    batch = index.shape[0]
    n = batch * embed_dim
    width = 128
    rows = n // width
    rows_per_w = rows // _NW

    table_flat = input.reshape(vocab * embed_dim)
    idx2d = index.astype(jnp.int32).reshape(rows, width)

    mesh = plsc.VectorSubcoreMesh(core_axis_name="c", subcore_axis_name="s",
                                  num_cores=_NC, num_subcores=_NS)
    body = functools.partial(_gather_body, rows_per_w, embed_dim)
    out = pl.kernel(
        body,
        out_type=jax.ShapeDtypeStruct((rows, width), jnp.float32),
        mesh=mesh,
        scratch_types=[
            pltpu.VMEM((rows_per_w, width), jnp.int32),
            pltpu.VMEM((rows_per_w, width), jnp.float32),
            pltpu.SemaphoreType.DMA,
        ],
    )(table_flat, idx2d)
    return out.reshape(batch, embed_dim)


# speedup vs baseline: 1.0012x; 1.0012x over previous
import functools

import jax
import jax.numpy as jnp
from jax import lax
from jax.experimental import pallas as pl
from jax.experimental.pallas import tpu as pltpu
from jax.experimental.pallas import tpu_sc as plsc

_NC = 2
_NS = 16
_NW = _NC * _NS
_LANES = 16


def _gather_body(n_per_w, embed_dim, table_hbm, idx_hbm, out_hbm,
                 idx_v, out_v, sem):
    wid = lax.axis_index("s") * _NC + lax.axis_index("c")
    base = wid * n_per_w

    pltpu.sync_copy(idx_hbm.at[pl.ds(base, n_per_w)], idx_v)

    lane_iota = lax.iota(jnp.int32, _LANES)
    period = embed_dim // _LANES

    def to_flat(g, carry):
        for c in range(period):
            i = g * period + c
            col0 = (c * _LANES) % embed_dim
            v = idx_v[pl.ds(i * _LANES, _LANES)]
            idx_v[pl.ds(i * _LANES, _LANES)] = (
                v * embed_dim + (lane_iota + col0))
        return carry

    lax.fori_loop(0, n_per_w // (_LANES * period), to_flat, 0, unroll=2)

    pltpu.async_copy(table_hbm.at[idx_v], out_v, sem).wait()

    pltpu.sync_copy(out_v, out_hbm.at[pl.ds(base, n_per_w)])


def kernel(input, index):
    vocab, embed_dim = input.shape
    batch = index.shape[0]
    n = batch * embed_dim
    n_per_w = n // _NW

    table_flat = input.reshape(vocab * embed_dim)
    idx_flat = index.astype(jnp.int32).reshape(n)

    mesh = plsc.VectorSubcoreMesh(core_axis_name="c", subcore_axis_name="s",
                                  num_cores=_NC, num_subcores=_NS)
    body = functools.partial(_gather_body, n_per_w, embed_dim)
    out = pl.kernel(
        body,
        out_type=jax.ShapeDtypeStruct((n,), jnp.float32),
        mesh=mesh,
        scratch_types=[
            pltpu.VMEM((n_per_w,), jnp.int32),
            pltpu.VMEM((n_per_w,), jnp.float32),
            pltpu.SemaphoreType.DMA,
        ],
    )(table_flat, idx_flat)
    return out.reshape(batch, embed_dim)
